# baseline (speedup 1.0000x reference)
import jax
import jax.numpy as jnp
from jax import lax
from jax.experimental import pallas as pl
from jax.experimental.pallas import tpu as pltpu
from jax.experimental.pallas import tpu_sc as plsc

VOCAB = 100000
D = 128
SENT = 50
BATCH = 4096
B = BATCH * SENT
NW = 32
ROWS_PER_W = B // NW
NBUF = 4
SENT_PER_CHUNK = 4
CHUNK = SENT_PER_CHUNK * SENT
NCHUNK = ROWS_PER_W // CHUNK
GLEAD = NBUF - 2
SCALE = float(D) ** 0.5
NLANE = D // 16


def _sc_body(idx_hbm, table_hbm, pe_hbm, out_hbm, idx_v, pe_v, *rest):
    bufs = rest[:NBUF]
    sems = rest[NBUF:2 * NBUF]
    osems = rest[2 * NBUF:3 * NBUF]
    shared = rest[3 * NBUF]
    ssem = rest[3 * NBUF + 1]

    sid = lax.axis_index("s")
    wid = sid * 2 + lax.axis_index("c")
    base = wid * ROWS_PER_W

    def start_gather(k):
        b = k % NBUF
        return pltpu.async_copy(
            table_hbm.at[idx_v.at[pl.ds(k * CHUNK, CHUNK)]], bufs[b], sems[b])

    def start_put(k, prev_put):
        b = k % NBUF
        if prev_put is not None:
            prev_put.wait()
        pltpu.async_copy(bufs[b], shared, ssem).wait()
        return pltpu.async_copy(
            shared, out_hbm.at[pl.ds(base + k * CHUNK, CHUNK)],
            osems[b])

    def compute(buf):
        def body_s(s, _):
            pe_vecs = [pe_v[s, pl.ds(v * 16, 16)] for v in range(NLANE)]
            for t in range(SENT_PER_CHUNK):
                r = t * SENT + s
                for v in range(NLANE):
                    sl = pl.ds(v * 16, 16)
                    buf[r, sl] = buf[r, sl] * SCALE + pe_vecs[v]
            return 0
        lax.fori_loop(0, SENT, body_s, 0, unroll=False)

    pltpu.sync_copy(idx_hbm.at[pl.ds(base, CHUNK)], idx_v.at[pl.ds(0, CHUNK)])
    in_flight = [None] * NBUF
    out_flight = [None] * NBUF
    in_flight[0] = start_gather(0)
    pltpu.sync_copy(idx_hbm.at[pl.ds(base + CHUNK, ROWS_PER_W - CHUNK)],
                    idx_v.at[pl.ds(CHUNK, ROWS_PER_W - CHUNK)])
    pltpu.sync_copy(pe_hbm, pe_v)
    for k in range(1, min(GLEAD + 1, NCHUNK)):
        in_flight[k % NBUF] = start_gather(k)
    last_put = None
    for k in range(NCHUNK):
        b = k % NBUF
        in_flight[b].wait()
        g = k + GLEAD + 1
        if g < NCHUNK:
            gb = g % NBUF
            in_flight[gb] = start_gather(g)
        compute(bufs[b])
        last_put = start_put(k, last_put)
    last_put.wait()


@jax.jit
def _run(idx_flat, table, pe):
    k = pl.kernel(
        _sc_body,
        out_type=jax.ShapeDtypeStruct((B, D), jnp.float32),
        mesh=plsc.VectorSubcoreMesh(core_axis_name="c", subcore_axis_name="s"),
        scratch_types=(
            [pltpu.VMEM((ROWS_PER_W,), jnp.int32),
             pltpu.VMEM((SENT, D), jnp.float32)]
            + [pltpu.VMEM((CHUNK, D), jnp.float32) for _ in range(NBUF)]
            + [pltpu.SemaphoreType.DMA for _ in range(2 * NBUF)]
            + [pltpu.VMEM_SHARED((CHUNK, D), jnp.float32),
               pltpu.SemaphoreType.DMA]
        ),
    )
    return k(idx_flat, table, pe)


def kernel(x, table, pos_encoding):
    idx_flat = x.reshape(-1).astype(jnp.int32)
    out = _run(idx_flat, table, pos_encoding)
    return out.reshape(BATCH, SENT, D)


# speedup vs baseline: 1.0014x; 1.0014x over previous
import jax
import jax.numpy as jnp
from jax import lax
from jax.experimental import pallas as pl
from jax.experimental.pallas import tpu as pltpu
from jax.experimental.pallas import tpu_sc as plsc

VOCAB = 100000
D = 128
SENT = 50
BATCH = 4096
B = BATCH * SENT
NW = 32
ROWS_PER_W = B // NW
NBUF = 4
SENT_PER_CHUNK = 4
CHUNK = SENT_PER_CHUNK * SENT
NCHUNK = ROWS_PER_W // CHUNK
GLEAD = NBUF - 2
SCALE = float(D) ** 0.5
NLANE = D // 16


def _sc_body(idx_hbm, table_hbm, pe_hbm, out_hbm, idx_v, pe_v, *rest):
    bufs = rest[:NBUF]
    sems = rest[NBUF:2 * NBUF]
    osems = rest[2 * NBUF:3 * NBUF]
    wid = lax.axis_index("s") * 2 + lax.axis_index("c")
    base = wid * ROWS_PER_W

    def start_gather(k):
        b = k % NBUF
        return pltpu.async_copy(
            table_hbm.at[idx_v.at[pl.ds(k * CHUNK, CHUNK)]], bufs[b], sems[b])

    def start_put(k):
        b = k % NBUF
        return pltpu.async_copy(
            bufs[b], out_hbm.at[pl.ds(base + k * CHUNK, CHUNK)], osems[b])

    def compute(buf):
        def body_s(s, _):
            pe_vecs = [pe_v[s, pl.ds(v * 16, 16)] for v in range(NLANE)]
            for t in range(SENT_PER_CHUNK):
                r = t * SENT + s
                for v in range(NLANE):
                    sl = pl.ds(v * 16, 16)
                    buf[r, sl] = buf[r, sl] * SCALE + pe_vecs[v]
            return 0
        lax.fori_loop(0, SENT, body_s, 0, unroll=False)

    pltpu.sync_copy(idx_hbm.at[pl.ds(base, CHUNK)], idx_v.at[pl.ds(0, CHUNK)])
    in_flight = [None] * NBUF
    out_flight = [None] * NBUF
    in_flight[0] = start_gather(0)
    pltpu.sync_copy(idx_hbm.at[pl.ds(base + CHUNK, ROWS_PER_W - CHUNK)],
                    idx_v.at[pl.ds(CHUNK, ROWS_PER_W - CHUNK)])
    pltpu.sync_copy(pe_hbm, pe_v)
    for k in range(1, min(GLEAD + 1, NCHUNK)):
        in_flight[k % NBUF] = start_gather(k)
    for k in range(NCHUNK):
        b = k % NBUF
        in_flight[b].wait()
        g = k + GLEAD + 1
        if g < NCHUNK:
            gb = g % NBUF
            if out_flight[gb] is not None:
                out_flight[gb].wait()
                out_flight[gb] = None
            in_flight[gb] = start_gather(g)
        compute(bufs[b])
        out_flight[b] = start_put(k)
    for b in range(NBUF):
        if out_flight[b] is not None:
            out_flight[b].wait()


@jax.jit
def _run(idx_flat, table, pe):
    k = pl.kernel(
        _sc_body,
        out_type=jax.ShapeDtypeStruct((B, D), jnp.float32),
        mesh=plsc.VectorSubcoreMesh(core_axis_name="c", subcore_axis_name="s"),
        scratch_types=(
            [pltpu.VMEM((ROWS_PER_W,), jnp.int32),
             pltpu.VMEM((SENT, D), jnp.float32)]
            + [pltpu.VMEM((CHUNK, D), jnp.float32) for _ in range(NBUF)]
            + [pltpu.SemaphoreType.DMA for _ in range(2 * NBUF)]
        ),
    )
    return k(idx_flat, table, pe)


def kernel(x, table, pos_encoding):
    idx_flat = x.reshape(-1).astype(jnp.int32)
    out = _run(idx_flat, table, pos_encoding)
    return out.reshape(BATCH, SENT, D)
